# baseline (speedup 1.0000x reference)
import functools

import jax
import jax.numpy as jnp
from jax import lax
from jax.experimental import pallas as pl
from jax.experimental.pallas import tpu as pltpu
from jax.experimental.pallas import tpu_sc as plsc

_NC, _NS, _L = 2, 16, 16
_NW = _NC * _NS
_CH = 128
_NB = 3
_SLACK = 2


def _sc_body(cnt_hbm, bf_hbm, pos_hbm, out_hbm,
             cnt_v, pos_v, buf0, buf1, buf2,
             sin0, sin1, sin2, sout0, sout1, sout2):
    n, d = bf_hbm.shape
    nseg = cnt_hbm.shape[0]
    ncol = d // _L
    rows_w = n // _NW
    nch = rows_w // _CH
    bufs = (buf0, buf1, buf2)
    sins = (sin0, sin1, sin2)
    souts = (sout0, sout1, sout2)
    wid = lax.axis_index("s") * _NC + lax.axis_index("c")
    row0 = wid * rows_w

    in_cp = [None] * nch
    out_cp = [None] * nch
    for k in range(min(_NB - _SLACK, nch)):
        in_cp[k] = pltpu.async_copy(
            bf_hbm.at[pl.ds(row0 + k * _CH, _CH)], bufs[k], sins[k])

    pltpu.sync_copy(cnt_hbm, cnt_v)
    pltpu.sync_copy(pos_hbm, pos_v)

    cnts = cnt_v[...]
    ends_vec = plsc.cumsum(cnts)
    total = ends_vec[nseg - 1]
    big = jnp.full((nseg,), jnp.int32(n + 1))
    zero = jnp.zeros((nseg,), jnp.int32)

    def process_chunk(buf, base):
        hi = jnp.clip(total - base, 0, _CH)

        def run_body(r):
            rg = jnp.full((nseg,), base + r)
            above = ends_vec > rg
            seg = jnp.sum((~above).astype(jnp.int32))
            run_end = jnp.min(jnp.where(above, ends_vec, big))
            run_start = jnp.max(jnp.where(above, zero, ends_vec))
            cnt_run = jnp.full((nseg,), run_end - run_start).astype(jnp.float32)
            inv = 1.0 / jnp.maximum(cnt_run, 1.0)
            pos_vecs = [pos_v[seg, pl.ds(cb * _L, _L)] for cb in range(ncol)]

            stop = jnp.minimum(run_end - base, hi)

            @plsc.parallel_loop(r, stop)
            def row_body(rr):
                for cb in range(ncol):
                    x = buf[rr, pl.ds(cb * _L, _L)]
                    buf[rr, pl.ds(cb * _L, _L)] = x * inv + pos_vecs[cb]

            return stop

        lax.while_loop(lambda r: r < hi, run_body, jnp.int32(0))

    for ci in range(nch):
        b = ci % _NB
        base = row0 + ci * _CH
        in_cp[ci].wait()
        process_chunk(bufs[b], base)
        out_cp[ci] = pltpu.async_copy(
            bufs[b], out_hbm.at[pl.ds(base, _CH)], souts[b])
        nxt = ci + _NB - _SLACK
        if nxt < nch:
            if nxt - _NB >= 0:
                out_cp[nxt - _NB].wait()
            in_cp[nxt] = pltpu.async_copy(
                bf_hbm.at[pl.ds(row0 + nxt * _CH, _CH)], bufs[nxt % _NB],
                sins[nxt % _NB])

    for ci in range(max(0, nch - _NB), nch):
        out_cp[ci].wait()


def kernel(eachimg_selected_box_nums, box_features, positions):
    n, d = box_features.shape
    nseg = eachimg_selected_box_nums.shape[0]
    cnt = eachimg_selected_box_nums.astype(jnp.int32)
    run = pl.kernel(
        _sc_body,
        out_type=jax.ShapeDtypeStruct((n, d), jnp.float32),
        mesh=plsc.VectorSubcoreMesh(
            core_axis_name="c", subcore_axis_name="s",
            num_cores=_NC, num_subcores=_NS,
        ),
        compiler_params=pltpu.CompilerParams(needs_layout_passes=False),
        scratch_types=(
            [pltpu.VMEM((nseg,), jnp.int32),
             pltpu.VMEM((nseg, d), jnp.float32)]
            + [pltpu.VMEM((_CH, d), jnp.float32)] * _NB
            + [pltpu.SemaphoreType.DMA] * (2 * _NB)
        ),
    )
    return run(cnt, box_features, positions)


# speedup vs baseline: 1.0764x; 1.0764x over previous
import functools

import jax
import jax.numpy as jnp
from jax import lax
from jax.experimental import pallas as pl
from jax.experimental.pallas import tpu as pltpu
from jax.experimental.pallas import tpu_sc as plsc

_NC, _NS, _L = 2, 16, 16
_NW = _NC * _NS
_CH = 64
_NB = 4
_SLACK = 2


def _sc_body(cnt_hbm, bf_hbm, pos_hbm, out_hbm,
             cnt_v, pos_v, buf0, buf1, buf2, buf3,
             sin0, sin1, sin2, sin3, sout0, sout1, sout2, sout3):
    n, d = bf_hbm.shape
    nseg = cnt_hbm.shape[0]
    ncol = d // _L
    rows_w = n // _NW
    nch = rows_w // _CH
    bufs = (buf0, buf1, buf2, buf3)
    sins = (sin0, sin1, sin2, sin3)
    souts = (sout0, sout1, sout2, sout3)
    wid = lax.axis_index("s") * _NC + lax.axis_index("c")
    row0 = wid * rows_w

    in_cp = [None] * nch
    out_cp = [None] * nch
    for k in range(min(_NB - _SLACK, nch)):
        in_cp[k] = pltpu.async_copy(
            bf_hbm.at[pl.ds(row0 + k * _CH, _CH)], bufs[k], sins[k])

    pltpu.sync_copy(cnt_hbm, cnt_v)
    pltpu.sync_copy(pos_hbm, pos_v)

    cnts = cnt_v[...]
    ends_vec = plsc.cumsum(cnts)
    total = ends_vec[nseg - 1]
    big = jnp.full((nseg,), jnp.int32(n + 1))
    zero = jnp.zeros((nseg,), jnp.int32)

    def process_chunk(buf, base):
        hi = jnp.clip(total - base, 0, _CH)

        def run_body(r):
            rg = jnp.full((nseg,), base + r)
            above = ends_vec > rg
            seg = jnp.sum((~above).astype(jnp.int32))
            run_end = jnp.min(jnp.where(above, ends_vec, big))
            run_start = jnp.max(jnp.where(above, zero, ends_vec))
            cnt_run = jnp.full((nseg,), run_end - run_start).astype(jnp.float32)
            inv = 1.0 / jnp.maximum(cnt_run, 1.0)
            pos_vecs = [pos_v[seg, pl.ds(cb * _L, _L)] for cb in range(ncol)]

            stop = jnp.minimum(run_end - base, hi)

            @plsc.parallel_loop(r, stop)
            def row_body(rr):
                for cb in range(ncol):
                    x = buf[rr, pl.ds(cb * _L, _L)]
                    buf[rr, pl.ds(cb * _L, _L)] = x * inv + pos_vecs[cb]

            return stop

        lax.while_loop(lambda r: r < hi, run_body, jnp.int32(0))

    for ci in range(nch):
        b = ci % _NB
        base = row0 + ci * _CH
        in_cp[ci].wait()
        process_chunk(bufs[b], base)
        out_cp[ci] = pltpu.async_copy(
            bufs[b], out_hbm.at[pl.ds(base, _CH)], souts[b])
        nxt = ci + _NB - _SLACK
        if nxt < nch:
            if nxt - _NB >= 0:
                out_cp[nxt - _NB].wait()
            in_cp[nxt] = pltpu.async_copy(
                bf_hbm.at[pl.ds(row0 + nxt * _CH, _CH)], bufs[nxt % _NB],
                sins[nxt % _NB])

    for ci in range(max(0, nch - _NB), nch):
        out_cp[ci].wait()


def kernel(eachimg_selected_box_nums, box_features, positions):
    n, d = box_features.shape
    nseg = eachimg_selected_box_nums.shape[0]
    cnt = eachimg_selected_box_nums.astype(jnp.int32)
    run = pl.kernel(
        _sc_body,
        out_type=jax.ShapeDtypeStruct((n, d), jnp.float32),
        mesh=plsc.VectorSubcoreMesh(
            core_axis_name="c", subcore_axis_name="s",
            num_cores=_NC, num_subcores=_NS,
        ),
        compiler_params=pltpu.CompilerParams(needs_layout_passes=False),
        scratch_types=(
            [pltpu.VMEM((nseg,), jnp.int32),
             pltpu.VMEM((nseg, d), jnp.float32)]
            + [pltpu.VMEM((_CH, d), jnp.float32)] * _NB
            + [pltpu.SemaphoreType.DMA] * (2 * _NB)
        ),
    )
    return run(cnt, box_features, positions)


# speedup vs baseline: 1.1622x; 1.0797x over previous
import jax
import jax.numpy as jnp
from jax import lax
from jax.experimental import pallas as pl
from jax.experimental.pallas import tpu as pltpu
from jax.experimental.pallas import tpu_sc as plsc

_NC, _NS, _L = 2, 16, 16
_NW = _NC * _NS
_CH = 128
_NB = 3
_SLACK = 1


def _sc_body(cnt_hbm, bf_hbm, pos_hbm, out_hbm,
             cnt_v, pos_v, buf0, buf1, buf2,
             sin0, sin1, sin2, sout0, sout1, sout2):
    n, d = bf_hbm.shape
    nseg = cnt_hbm.shape[0]
    ncol = d // _L
    rows_w = n // _NW
    nch = rows_w // _CH
    bufs = (buf0, buf1, buf2)
    sins = (sin0, sin1, sin2)
    souts = (sout0, sout1, sout2)
    wid = lax.axis_index("s") * _NC + lax.axis_index("c")
    row0 = wid * rows_w

    in_cp = [None] * nch
    out_cp = [None] * nch
    for k in range(min(_NB - _SLACK, nch)):
        in_cp[k] = pltpu.async_copy(
            bf_hbm.at[pl.ds(row0 + k * _CH, _CH)], bufs[k], sins[k])

    pltpu.sync_copy(cnt_hbm, cnt_v)
    pltpu.sync_copy(pos_hbm, pos_v)

    cnts = cnt_v[...]
    ends_vec = plsc.cumsum(cnts)
    total = ends_vec[nseg - 1]
    big = jnp.full((nseg,), jnp.int32(n + 1))
    zero = jnp.zeros((nseg,), jnp.int32)

    def process_chunk(buf, base):
        hi = jnp.clip(total - base, 0, _CH)

        def run_body(r):
            rg = jnp.full((nseg,), base + r)
            above = ends_vec > rg
            seg = jnp.sum((~above).astype(jnp.int32))
            run_end = jnp.min(jnp.where(above, ends_vec, big))
            run_start = jnp.max(jnp.where(above, zero, ends_vec))
            cnt_run = jnp.full((nseg,), run_end - run_start).astype(jnp.float32)
            inv = 1.0 / jnp.maximum(cnt_run, 1.0)
            pos_vecs = [pos_v[seg, pl.ds(cb * _L, _L)] for cb in range(ncol)]

            stop = jnp.minimum(run_end - base, hi)

            @plsc.parallel_loop(r, stop)
            def row_body(rr):
                for cb in range(ncol):
                    x = buf[rr, pl.ds(cb * _L, _L)]
                    buf[rr, pl.ds(cb * _L, _L)] = x * inv + pos_vecs[cb]

            return stop

        lax.while_loop(lambda r: r < hi, run_body, jnp.int32(0))

    for ci in range(nch):
        b = ci % _NB
        base = row0 + ci * _CH
        in_cp[ci].wait()
        process_chunk(bufs[b], base)
        out_cp[ci] = pltpu.async_copy(
            bufs[b], out_hbm.at[pl.ds(base, _CH)], souts[b])
        nxt = ci + _NB - _SLACK
        if nxt < nch:
            if nxt - _NB >= 0:
                out_cp[nxt - _NB].wait()
            in_cp[nxt] = pltpu.async_copy(
                bf_hbm.at[pl.ds(row0 + nxt * _CH, _CH)], bufs[nxt % _NB],
                sins[nxt % _NB])

    for ci in range(max(0, nch - _NB), nch):
        out_cp[ci].wait()


def kernel(eachimg_selected_box_nums, box_features, positions):
    n, d = box_features.shape
    nseg = eachimg_selected_box_nums.shape[0]
    cnt = eachimg_selected_box_nums.astype(jnp.int32)
    run = pl.kernel(
        _sc_body,
        out_type=jax.ShapeDtypeStruct((n, d), jnp.float32),
        mesh=plsc.VectorSubcoreMesh(
            core_axis_name="c", subcore_axis_name="s",
            num_cores=_NC, num_subcores=_NS,
        ),
        compiler_params=pltpu.CompilerParams(needs_layout_passes=False),
        scratch_types=(
            [pltpu.VMEM((nseg,), jnp.int32),
             pltpu.VMEM((nseg, d), jnp.float32)]
            + [pltpu.VMEM((_CH, d), jnp.float32)] * _NB
            + [pltpu.SemaphoreType.DMA] * (2 * _NB)
        ),
    )
    return run(cnt, box_features, positions)
